# baseline (speedup 1.0000x reference)
import functools

import jax
import jax.numpy as jnp
from jax import lax
from jax.experimental import pallas as pl
from jax.experimental.pallas import tpu as pltpu
from jax.experimental.pallas import tpu_sc as plsc

B = 4096
L = 200
D = 128
VOCAB = 100000
NC = 2
NS = 16
LANES = 16
NW = NC * NS
BPW = B // NW
DV = D // LANES
PREP_BLK = 512
QUARTS = ((0, 56), (56, 48), (104, 48), (152, 48))
NQ = 4
QROWS = 56
OGRP = 8


def _prep_body(ids_ref, trow_ref, idc_ref):
    x = ids_ref[...]
    lens = jnp.sum((x != 0).astype(jnp.int32), axis=1, keepdims=True)
    lens_bits = lax.bitcast_convert_type(
        jnp.broadcast_to(lens.astype(jnp.float32), (PREP_BLK, LANES)), jnp.int32
    )
    r0 = lax.bitcast_convert_type(trow_ref[...], jnp.int32)[0:1, :]
    r0a = jnp.broadcast_to(r0[:, : D // 2], (PREP_BLK, D // 2))
    r0b = jnp.broadcast_to(r0[:, D // 2 :], (PREP_BLK, D // 2))
    pos = lax.broadcasted_iota(jnp.int32, (PREP_BLK, L), 1)
    c = jnp.where(pos < lens, x, 0)

    def z(n):
        return jnp.zeros((PREP_BLK, n), jnp.int32)

    parts = [
        c[:, 0:56], lens_bits, z(128 - 56 - LANES),
        c[:, 56:104], r0a, z(128 - 48 - D // 2),
        c[:, 104:152], r0b, z(128 - 48 - D // 2),
        c[:, 152:200], z(128 - 48),
    ]
    idc_ref[...] = jnp.concatenate(parts, axis=1).reshape(PREP_BLK * NQ, 128)


def _prep(input_ids, table):
    return pl.pallas_call(
        _prep_body,
        out_shape=jax.ShapeDtypeStruct((B * NQ, 128), jnp.int32),
        grid=(B // PREP_BLK,),
        in_specs=[
            pl.BlockSpec((PREP_BLK, L), lambda i: (i, 0)),
            pl.BlockSpec((8, D), lambda i: (0, 0)),
        ],
        out_specs=pl.BlockSpec((PREP_BLK * NQ, 128), lambda i: (i, 0)),
    )(input_ids, table)


def _make_sc_kernel():
    mesh = plsc.VectorSubcoreMesh(core_axis_name="c", subcore_axis_name="s")

    @functools.partial(
        pl.kernel,
        mesh=mesh,
        out_type=jax.ShapeDtypeStruct((B, D), jnp.float32),
        scratch_types=[
            pltpu.VMEM((BPW * NQ, 128), jnp.int32),
            pltpu.VMEM((2, QROWS, D), jnp.float32),
            pltpu.VMEM((6, 48, D), jnp.float32),
            pltpu.VMEM((2, OGRP, D), jnp.float32),
            pltpu.SemaphoreType.DMA,
            pltpu.SemaphoreType.DMA,
            pltpu.SemaphoreType.DMA,
            pltpu.SemaphoreType.DMA,
            pltpu.SemaphoreType.DMA,
            pltpu.SemaphoreType.DMA,
            pltpu.SemaphoreType.DMA,
            pltpu.SemaphoreType.DMA,
            pltpu.SemaphoreType.DMA,
            pltpu.SemaphoreType.DMA,
        ],
    )
    def k(ids_hbm, table_hbm, out_hbm,
          ids_v, rows_a, rows_b, outb_v,
          sem0, sem1, sem2, sem3, sem4, sem5, sem6, sem7, osem0, osem1):
        wid = lax.axis_index("s") * NC + lax.axis_index("c")
        base = wid * BPW
        pltpu.sync_copy(ids_hbm.at[pl.ds(base * NQ, BPW * NQ)], ids_v)

        sems = (sem0, sem1, sem2, sem3, sem4, sem5, sem6, sem7)

        def slot_buf(par, q):
            if q == 0:
                return rows_a, par
            return rows_b, 3 * par + (q - 1)

        def issue(s, par, q):
            n = QUARTS[q][1]
            buf, bi = slot_buf(par, q)
            pltpu.make_async_copy(
                table_hbm.at[ids_v.at[s * NQ + q, pl.ds(0, n)]],
                buf.at[bi, pl.ds(0, n)],
                sems[NQ * par + q],
            ).start()

        def wait(par, q):
            n = QUARTS[q][1]
            buf, bi = slot_buf(par, q)
            pltpu.make_async_copy(
                table_hbm.at[pl.ds(0, n)],
                buf.at[bi, pl.ds(0, n)],
                sems[NQ * par + q],
            ).wait()

        def accumulate(par, q, acc):
            buf, bi = slot_buf(par, q)
            n_iters = QUARTS[q][1] // 4

            def body(t, acc):
                r0 = 4 * t
                for dr in range(4):
                    r = r0 + dr
                    acc = tuple(
                        acc[kk] + buf[bi, r, pl.ds(kk * 16, 16)]
                        for kk in range(DV)
                    )
                return acc

            return lax.fori_loop(0, n_iters, body, acc)

        def finalize(s, orow, obuf, acc):
            len_f = lax.bitcast_convert_type(
                ids_v[s * NQ, pl.ds(56, LANES)], jnp.float32
            )
            ninv = jnp.full((LANES,), float(L), jnp.float32) - len_f
            for kk in range(DV):
                qrow = 1 if kk < DV // 2 else 2
                off = 48 + 16 * (kk % (DV // 2))
                r0 = lax.bitcast_convert_type(
                    ids_v[s * NQ + qrow, pl.ds(off, 16)], jnp.float32
                )
                outb_v[obuf, orow, pl.ds(kk * 16, 16)] = (
                    (acc[kk] - ninv * r0) / len_f
                )

        zero = jnp.zeros((LANES,), jnp.float32)
        zeros8 = tuple(zero for _ in range(DV))

        for par in range(2):
            for q in range(NQ):
                issue(par, par, q)

        osems = (osem0, osem1)

        def ogrp_pair_body(g2, carry):
            for p in range(2):
                g = 2 * g2 + p
                s0 = OGRP * g

                @pl.when(g2 >= 1)
                def _():
                    pltpu.make_async_copy(
                        outb_v.at[p],
                        out_hbm.at[pl.ds(base, OGRP)],
                        osems[p],
                    ).wait()

                for half in range(OGRP // 2):
                    for par in range(2):
                        ds_ = 2 * half + par
                        s = s0 + ds_
                        acc = zeros8
                        for q in range(NQ):
                            wait(par, q)
                            acc = accumulate(par, q, acc)

                            @pl.when(s + 2 < BPW)
                            def _():
                                issue(s + 2, par, q)

                        finalize(s, ds_, p, acc)

                pltpu.make_async_copy(
                    outb_v.at[p],
                    out_hbm.at[pl.ds(base + s0, OGRP)],
                    osems[p],
                ).start()
            return carry

        lax.fori_loop(0, BPW // OGRP // 2, ogrp_pair_body, 0)

        for p in range(2):
            pltpu.make_async_copy(
                outb_v.at[p],
                out_hbm.at[pl.ds(base, OGRP)],
                osems[p],
            ).wait()

    return k


_sc_kernel = _make_sc_kernel()


@jax.jit
def kernel(input_ids, table):
    ids = input_ids.astype(jnp.int32)
    ids_q = _prep(ids, table)
    return _sc_kernel(ids_q, table)


# speedup vs baseline: 1.0026x; 1.0026x over previous
import functools

import jax
import jax.numpy as jnp
from jax import lax
from jax.experimental import pallas as pl
from jax.experimental.pallas import tpu as pltpu
from jax.experimental.pallas import tpu_sc as plsc

B = 4096
L = 200
D = 128
VOCAB = 100000
NC = 2
NS = 16
LANES = 16
NW = NC * NS
BPW = B // NW
DV = D // LANES
PREP_BLK = 512
LP = 256
HALVES = ((0, 0, 104), (104, 128, 96))
LEN_OFF = 104
OGRP = 8


def _prep_body(ids_ref, idc_ref):
    x = ids_ref[...]
    lens = jnp.sum((x != 0).astype(jnp.int32), axis=1, keepdims=True)
    lens_bits = lax.bitcast_convert_type(
        jnp.broadcast_to(lens.astype(jnp.float32), (PREP_BLK, LANES)), jnp.int32
    )
    pos = lax.broadcasted_iota(jnp.int32, (PREP_BLK, L), 1)
    c = jnp.where(pos < lens, x, 0)
    parts = [
        c[:, 0:104],
        lens_bits,
        jnp.zeros((PREP_BLK, 128 - 104 - LANES), jnp.int32),
        c[:, 104:200],
        jnp.zeros((PREP_BLK, LP - 128 - 96), jnp.int32),
    ]
    idc_ref[...] = jnp.concatenate(parts, axis=1)


def _prep(input_ids):
    return pl.pallas_call(
        _prep_body,
        out_shape=jax.ShapeDtypeStruct((B, LP), jnp.int32),
        grid=(B // PREP_BLK,),
        in_specs=[pl.BlockSpec((PREP_BLK, L), lambda i: (i, 0))],
        out_specs=pl.BlockSpec((PREP_BLK, LP), lambda i: (i, 0)),
    )(input_ids)


def _make_sc_kernel():
    mesh = plsc.VectorSubcoreMesh(core_axis_name="c", subcore_axis_name="s")

    @functools.partial(
        pl.kernel,
        mesh=mesh,
        out_type=jax.ShapeDtypeStruct((B, D), jnp.float32),
        scratch_types=[
            pltpu.VMEM((BPW, LP), jnp.int32),
            pltpu.VMEM((4, 104, D), jnp.float32),
            pltpu.VMEM((8, D), jnp.float32),
            pltpu.VMEM((2, OGRP, D), jnp.float32),
            pltpu.SemaphoreType.DMA,
            pltpu.SemaphoreType.DMA,
            pltpu.SemaphoreType.DMA,
            pltpu.SemaphoreType.DMA,
            pltpu.SemaphoreType.DMA,
            pltpu.SemaphoreType.DMA,
        ],
    )
    def k(ids_hbm, table_hbm, out_hbm,
          ids_v, rows_v, row0_v, outb_v,
          sem0, sem1, sem2, sem3, osem0, osem1):
        wid = lax.axis_index("s") * NC + lax.axis_index("c")
        base = wid * BPW
        pltpu.sync_copy(ids_hbm.at[pl.ds(base, BPW)], ids_v)
        pltpu.sync_copy(table_hbm.at[pl.ds(0, 8)], row0_v)

        sems = (sem0, sem1, sem2, sem3)

        def issue(s, par, h):
            _, woff, n = HALVES[h]
            slot = 2 * par + h
            pltpu.make_async_copy(
                table_hbm.at[ids_v.at[s, pl.ds(woff, n)]],
                rows_v.at[slot, pl.ds(0, n)],
                sems[slot],
            ).start()

        def wait(par, h):
            n = HALVES[h][2]
            slot = 2 * par + h
            pltpu.make_async_copy(
                table_hbm.at[pl.ds(0, n)],
                rows_v.at[slot, pl.ds(0, n)],
                sems[slot],
            ).wait()

        def accumulate(par, h, acc):
            slot = 2 * par + h
            n_iters = HALVES[h][2] // 4

            def body(t, acc):
                r0 = 4 * t
                for dr in range(4):
                    r = r0 + dr
                    acc = tuple(
                        acc[kk] + rows_v[slot, r, pl.ds(kk * 16, 16)]
                        for kk in range(DV)
                    )
                return acc

            return lax.fori_loop(0, n_iters, body, acc)

        def finalize(s, orow, obuf, acc):
            len_f = lax.bitcast_convert_type(
                ids_v[s, pl.ds(LEN_OFF, LANES)], jnp.float32
            )
            ninv = jnp.full((LANES,), float(L), jnp.float32) - len_f
            for kk in range(DV):
                r0 = row0_v[0, pl.ds(kk * 16, 16)]
                outb_v[obuf, orow, pl.ds(kk * 16, 16)] = (
                    (acc[kk] - ninv * r0) / len_f
                )

        zero = jnp.zeros((LANES,), jnp.float32)
        zeros8 = tuple(zero for _ in range(DV))

        for par in range(2):
            for h in range(2):
                issue(par, par, h)

        osems = (osem0, osem1)

        def ogrp_pair_body(g2, carry):
            for p in range(2):
                g = 2 * g2 + p
                s0 = OGRP * g

                @pl.when(g2 >= 1)
                def _():
                    pltpu.make_async_copy(
                        outb_v.at[p],
                        out_hbm.at[pl.ds(base, OGRP)],
                        osems[p],
                    ).wait()

                for half in range(OGRP // 2):
                    for par in range(2):
                        ds_ = 2 * half + par
                        s = s0 + ds_
                        acc = zeros8
                        for h in range(2):
                            wait(par, h)
                            acc = accumulate(par, h, acc)

                            @pl.when(s + 2 < BPW)
                            def _():
                                issue(s + 2, par, h)

                        finalize(s, ds_, p, acc)

                pltpu.make_async_copy(
                    outb_v.at[p],
                    out_hbm.at[pl.ds(base + s0, OGRP)],
                    osems[p],
                ).start()
            return carry

        lax.fori_loop(0, BPW // OGRP // 2, ogrp_pair_body, 0)

        for p in range(2):
            pltpu.make_async_copy(
                outb_v.at[p],
                out_hbm.at[pl.ds(base, OGRP)],
                osems[p],
            ).wait()

    return k


_sc_kernel = _make_sc_kernel()


@jax.jit
def kernel(input_ids, table):
    ids = input_ids.astype(jnp.int32)
    ids_q = _prep(ids)
    return _sc_kernel(ids_q, table)
